# baseline (speedup 1.0000x reference)
import functools

import jax
import jax.numpy as jnp
from jax import lax
from jax.experimental import pallas as pl
from jax.experimental.pallas import tpu as pltpu
from jax.experimental.pallas import tpu_sc as plsc

EMBED = 64
NC = 2
NS = 16
NW = NC * NS
CHUNK = 128
LANES = 16
NSETS = 2
ROW_OFF = (0, 12, 43, 67)
TOT_ROWS = 74
OUT_D = 5 * EMBED


def _sc_body(per_w, n_chunks, idx_hbm, wm, wd, wh, ww, wpos, out_hbm,
             idx_v, asm_v, tab_v, crow_v, sem):
    wid = lax.axis_index("s") * NC + lax.axis_index("c")
    base_w = wid * per_w
    tables = [wm, wd, wh, ww]

    pltpu.sync_copy(idx_hbm.at[:, pl.ds(base_w, per_w)], idx_v)
    for t in range(4):
        nwords = (ROW_OFF[t + 1] - ROW_OFF[t]) * EMBED if t < 3 \
            else (TOT_ROWS - ROW_OFF[3]) * EMBED
        pltpu.sync_copy(tables[t],
                        tab_v.at[pl.ds(ROW_OFF[t] * EMBED, nwords)])
    pltpu.sync_copy(wpos.at[pl.ds(0, EMBED)], crow_v)

    crow = [crow_v[pl.ds(l * LANES, LANES)] for l in range(EMBED // LANES)]

    def fill_body(r, carry):
        for l in range(EMBED // LANES):
            asm_v[pl.ds(r * OUT_D + 4 * EMBED + l * LANES, LANES)] = crow[l]
        return carry

    lax.fori_loop(0, NSETS * CHUNK, fill_body, 0)

    iota = lax.iota(jnp.int32, LANES)
    iota_out = iota * OUT_D

    def chunk_body(c, carry):
        sbase = (c % NSETS) * CHUNK

        @pl.when(c >= NSETS)
        def _():
            pltpu.make_async_copy(
                asm_v.at[pl.ds(0, CHUNK * OUT_D)],
                out_hbm.at[pl.ds(0, CHUNK * OUT_D)], sem).wait()

        for t in range(4):
            toff = ROW_OFF[t] * EMBED

            def grp_body(g, carry2):
                pos0 = c * CHUNK + g * LANES
                idx16 = idx_v[t, pl.ds(pos0, LANES)]
                addr = idx16 * EMBED + toff
                dst = iota_out + ((sbase + g * LANES) * OUT_D + t * EMBED)
                for col in range(EMBED):
                    vals = plsc.load_gather(tab_v, [addr + col])
                    plsc.store_scatter(asm_v, [dst + col], vals)
                return carry2

            lax.fori_loop(0, CHUNK // LANES, grp_body, 0)

        pltpu.async_copy(
            asm_v.at[pl.ds(sbase * OUT_D, CHUNK * OUT_D)],
            out_hbm.at[pl.ds((base_w + c * CHUNK) * OUT_D, CHUNK * OUT_D)],
            sem)
        return carry

    lax.fori_loop(0, n_chunks, chunk_body, 0)

    for _ in range(min(NSETS, n_chunks)):
        pltpu.make_async_copy(
            asm_v.at[pl.ds(0, CHUNK * OUT_D)],
            out_hbm.at[pl.ds(0, CHUNK * OUT_D)], sem).wait()


def kernel(context, y, W_month, W_day, W_hour, W_dow, W_pos):
    del y
    Bc, Nc, _ = context.shape
    M = Bc * Nc
    assert M % (NW * CHUNK) == 0
    per_w = M // NW
    n_chunks = per_w // CHUNK
    idxs = context.reshape(M, 5).astype(jnp.int32)[:, 1:5].T

    mesh = plsc.VectorSubcoreMesh(core_axis_name="c", subcore_axis_name="s")
    f32 = jnp.float32
    run = pl.kernel(
        functools.partial(_sc_body, per_w, n_chunks),
        out_type=jax.ShapeDtypeStruct((M * OUT_D,), f32),
        mesh=mesh,
        scratch_types=[
            pltpu.VMEM((4, per_w), jnp.int32),
            pltpu.VMEM((NSETS * CHUNK * OUT_D,), f32),
            pltpu.VMEM((TOT_ROWS * EMBED,), f32),
            pltpu.VMEM((EMBED,), f32),
            pltpu.SemaphoreType.DMA,
        ],
        compiler_params=pltpu.CompilerParams(use_tc_tiling_on_sc=False,
                                             needs_layout_passes=False),
    )
    out = run(jnp.asarray(idxs),
              W_month.astype(f32).reshape(-1), W_day.astype(f32).reshape(-1),
              W_hour.astype(f32).reshape(-1), W_dow.astype(f32).reshape(-1),
              W_pos.astype(f32).reshape(-1))
    return out.reshape(Bc, Nc, OUT_D)


# speedup vs baseline: 2.1134x; 2.1134x over previous
import functools

import jax
import jax.numpy as jnp
from jax import lax
from jax.experimental import pallas as pl
from jax.experimental.pallas import tpu as pltpu
from jax.experimental.pallas import tpu_sc as plsc

EMBED = 64
NC = 2
NS = 16
NW = NC * NS
CHUNK = 128
LANES = 16
NSETS = 2
ROW_OFF = (0, 12, 43, 67)
TOT_ROWS = 74
OUT_D = 5 * EMBED
UNROLL = 2


def _sc_body(per_w, n_chunks, ctx_hbm, wm, wd, wh, ww, wpos, out_hbm,
             ctx_v, asm_v, tab_v, crow_v, sem):
    wid = lax.axis_index("s") * NC + lax.axis_index("c")
    base_w = wid * per_w
    tables = [wm, wd, wh, ww]
    nrows = (12, 31, 24, 7)

    pltpu.sync_copy(ctx_hbm.at[pl.ds(base_w * 5, per_w * 5)], ctx_v)
    for t in range(4):
        pltpu.sync_copy(tables[t],
                        tab_v.at[pl.ds(ROW_OFF[t] * EMBED, nrows[t] * EMBED)])
    pltpu.sync_copy(wpos.at[pl.ds(0, EMBED)], crow_v)

    crow = [crow_v[pl.ds(l * LANES, LANES)] for l in range(EMBED // LANES)]

    def fill_body(r, carry):
        for l in range(EMBED // LANES):
            asm_v[pl.ds(r * OUT_D + 4 * EMBED + l * LANES, LANES)] = crow[l]
        return carry

    lax.fori_loop(0, NSETS * CHUNK, fill_body, 0)

    iota = lax.iota(jnp.int32, LANES)

    def chunk_body(c, carry):
        sbase = (c % NSETS) * CHUNK

        @pl.when(c >= NSETS)
        def _():
            pltpu.make_async_copy(
                asm_v.at[pl.ds(0, CHUNK * OUT_D)],
                out_hbm.at[pl.ds(0, CHUNK * OUT_D)], sem).wait()

        def pos_body(i, carry2):
            for u in range(UNROLL):
                p = i * UNROLL + u
                src = c * CHUNK + p
                dbase = (sbase + p) * OUT_D
                v16 = plsc.load_gather(ctx_v, [iota + (src * 5)])
                for t in range(4):
                    r = v16[t + 1]
                    rb = r * EMBED + ROW_OFF[t] * EMBED
                    db = dbase + t * EMBED
                    for l in range(EMBED // LANES):
                        asm_v[pl.ds(db + l * LANES, LANES)] = \
                            tab_v[pl.ds(rb + l * LANES, LANES)]
            return carry2

        lax.fori_loop(0, CHUNK // UNROLL, pos_body, 0)

        pltpu.async_copy(
            asm_v.at[pl.ds(sbase * OUT_D, CHUNK * OUT_D)],
            out_hbm.at[pl.ds((base_w + c * CHUNK) * OUT_D, CHUNK * OUT_D)],
            sem)
        return carry

    lax.fori_loop(0, n_chunks, chunk_body, 0)

    for _ in range(min(NSETS, n_chunks)):
        pltpu.make_async_copy(
            asm_v.at[pl.ds(0, CHUNK * OUT_D)],
            out_hbm.at[pl.ds(0, CHUNK * OUT_D)], sem).wait()


def kernel(context, y, W_month, W_day, W_hour, W_dow, W_pos):
    del y
    Bc, Nc, _ = context.shape
    M = Bc * Nc
    assert M % (NW * CHUNK) == 0
    per_w = M // NW
    n_chunks = per_w // CHUNK
    ctx = context.reshape(M * 5).astype(jnp.int32)

    mesh = plsc.VectorSubcoreMesh(core_axis_name="c", subcore_axis_name="s")
    f32 = jnp.float32
    run = pl.kernel(
        functools.partial(_sc_body, per_w, n_chunks),
        out_type=jax.ShapeDtypeStruct((M * OUT_D,), f32),
        mesh=mesh,
        scratch_types=[
            pltpu.VMEM((per_w * 5,), jnp.int32),
            pltpu.VMEM((NSETS * CHUNK * OUT_D,), f32),
            pltpu.VMEM((TOT_ROWS * EMBED,), f32),
            pltpu.VMEM((EMBED,), f32),
            pltpu.SemaphoreType.DMA,
        ],
        compiler_params=pltpu.CompilerParams(use_tc_tiling_on_sc=False,
                                             needs_layout_passes=False),
    )
    out = run(ctx,
              W_month.astype(f32).reshape(-1), W_day.astype(f32).reshape(-1),
              W_hour.astype(f32).reshape(-1), W_dow.astype(f32).reshape(-1),
              W_pos.astype(f32).reshape(-1))
    return out.reshape(Bc, Nc, OUT_D)


# speedup vs baseline: 3.0533x; 1.4447x over previous
import functools

import jax
import jax.numpy as jnp
from jax import lax
from jax.experimental import pallas as pl
from jax.experimental.pallas import tpu as pltpu
from jax.experimental.pallas import tpu_sc as plsc

EMBED = 64
NC = 2
NS = 16
NW = NC * NS
CHUNK = 128
LANES = 16
NSETS = 2
ROW_OFF = (0, 12, 43, 67)
TOT_ROWS = 74
OUT_D = 5 * EMBED
UNROLL = 4


def _sc_body(per_w, n_chunks, ctx_hbm, wm, wd, wh, ww, wpos, out_hbm,
             ctx_v, asm_v, tab_v, crow_v, sem):
    wid = lax.axis_index("s") * NC + lax.axis_index("c")
    base_w = wid * per_w
    tables = [wm, wd, wh, ww]
    nrows = (12, 31, 24, 7)

    pltpu.sync_copy(ctx_hbm.at[pl.ds(base_w * 5, per_w * 5)], ctx_v)
    for t in range(4):
        pltpu.sync_copy(tables[t],
                        tab_v.at[pl.ds(ROW_OFF[t] * EMBED, nrows[t] * EMBED)])
    pltpu.sync_copy(wpos.at[pl.ds(0, EMBED)], crow_v)

    crow = [crow_v[pl.ds(l * LANES, LANES)] for l in range(EMBED // LANES)]

    def fill_body(r, carry):
        for l in range(EMBED // LANES):
            asm_v[pl.ds(r * OUT_D + 4 * EMBED + l * LANES, LANES)] = crow[l]
        return carry

    lax.fori_loop(0, NSETS * CHUNK, fill_body, 0)

    iota = lax.iota(jnp.int32, LANES)

    def chunk_body(c, carry):
        sbase = (c % NSETS) * CHUNK

        @pl.when(c >= NSETS)
        def _():
            pltpu.make_async_copy(
                asm_v.at[pl.ds(0, CHUNK * OUT_D)],
                out_hbm.at[pl.ds(0, CHUNK * OUT_D)], sem).wait()

        @plsc.parallel_loop(0, CHUNK, step=1, unroll=UNROLL)
        def pos_body(p):
            src = c * CHUNK + p
            dbase = (sbase + p) * OUT_D
            v16 = plsc.load_gather(ctx_v, [iota + (src * 5)])
            for t in range(4):
                r = v16[t + 1]
                rb = r * EMBED + ROW_OFF[t] * EMBED
                db = dbase + t * EMBED
                for l in range(EMBED // LANES):
                    asm_v[pl.ds(db + l * LANES, LANES)] = \
                        tab_v[pl.ds(rb + l * LANES, LANES)]

        pltpu.async_copy(
            asm_v.at[pl.ds(sbase * OUT_D, CHUNK * OUT_D)],
            out_hbm.at[pl.ds((base_w + c * CHUNK) * OUT_D, CHUNK * OUT_D)],
            sem)
        return carry

    lax.fori_loop(0, n_chunks, chunk_body, 0)

    for _ in range(min(NSETS, n_chunks)):
        pltpu.make_async_copy(
            asm_v.at[pl.ds(0, CHUNK * OUT_D)],
            out_hbm.at[pl.ds(0, CHUNK * OUT_D)], sem).wait()


def kernel(context, y, W_month, W_day, W_hour, W_dow, W_pos):
    del y
    Bc, Nc, _ = context.shape
    M = Bc * Nc
    assert M % (NW * CHUNK) == 0
    per_w = M // NW
    n_chunks = per_w // CHUNK
    ctx = context.reshape(M * 5).astype(jnp.int32)

    mesh = plsc.VectorSubcoreMesh(core_axis_name="c", subcore_axis_name="s")
    f32 = jnp.float32
    run = pl.kernel(
        functools.partial(_sc_body, per_w, n_chunks),
        out_type=jax.ShapeDtypeStruct((M * OUT_D,), f32),
        mesh=mesh,
        scratch_types=[
            pltpu.VMEM((per_w * 5,), jnp.int32),
            pltpu.VMEM((NSETS * CHUNK * OUT_D,), f32),
            pltpu.VMEM((TOT_ROWS * EMBED,), f32),
            pltpu.VMEM((EMBED,), f32),
            pltpu.SemaphoreType.DMA,
        ],
        compiler_params=pltpu.CompilerParams(use_tc_tiling_on_sc=False,
                                             needs_layout_passes=False),
    )
    out = run(ctx,
              W_month.astype(f32).reshape(-1), W_day.astype(f32).reshape(-1),
              W_hour.astype(f32).reshape(-1), W_dow.astype(f32).reshape(-1),
              W_pos.astype(f32).reshape(-1))
    return out.reshape(Bc, Nc, OUT_D)
